# Initial kernel scaffold; baseline (speedup 1.0000x reference)
import jax
import jax.numpy as jnp
from jax.experimental import pallas as pl


def kernel(ent_table, rel_table, W1, Wr1, a_src1, a_dst1, a_rel1, W2, Wr2, a_src2, a_dst2, a_rel2, ent_ids, rel_ids, edge_index, cls_idx):
    raise NotImplementedError("write your pallas kernel here")


# speedup vs baseline: 28.3237x; 28.3237x over previous
import dataclasses
import functools

import numpy as np
import jax
import jax.numpy as jnp
from jax import lax
from jax.experimental import pallas as pl
from jax.experimental.pallas import tpu as pltpu
from jax.experimental.pallas import tpu_sc as plsc

N_NODES = 10000
N_EDGES = 320000
N_REL = 1000
HID = 128
HEADS = 4
D_HEAD = HID // HEADS
NEG_SLOPE = 0.2

NC = 2
NS = 16
LANES = 16
NW = NC * NS

HHALF = HID // NC
EK = 80
E_PER_S = N_EDGES // NS
N_CHUNKS = E_PER_S // EK

NPAD = 10240
GPW = NPAD // NW
ROWS_PER_SUB = NPAD // NS


def _vector_mesh():
    return plsc.VectorSubcoreMesh(core_axis_name="c", subcore_axis_name="s")


def _sc_compiler_params():
    cp = pltpu.CompilerParams()
    fields = pltpu.CompilerParams.__dataclass_fields__
    if "needs_layout_passes" in fields:
        cp = dataclasses.replace(cp, needs_layout_passes=False)
    if "use_tc_tiling_on_sc" in fields:
        cp = dataclasses.replace(cp, use_tc_tiling_on_sc=False)
    return cp


@jax.jit
def _sc_gather_rows(table, idx):
    n_sub = GPW // EK

    @functools.partial(
        pl.kernel,
        out_type=jax.ShapeDtypeStruct((NPAD, HID), jnp.float32),
        mesh=_vector_mesh(),
        scratch_types=[
            pltpu.VMEM((EK,), jnp.int32),
            pltpu.VMEM((EK, HID), jnp.float32),
            pltpu.SemaphoreType.DMA,
        ],
    )
    def k(table_hbm, idx_hbm, out_hbm, idx_v, rows_v, sem):
        wid = lax.axis_index("s") * NC + lax.axis_index("c")

        @pl.loop(0, n_sub)
        def _(g):
            base = wid * GPW + g * EK
            pltpu.sync_copy(idx_hbm.at[pl.ds(base, EK)], idx_v)
            pltpu.async_copy(table_hbm.at[idx_v], rows_v, sem).wait()
            pltpu.sync_copy(rows_v, out_hbm.at[pl.ds(base, EK)])

    return k(table, idx)


@jax.jit
def _sc_edge_pass(src, dst, rel, whs_flat, eler_flat, ee_flat, z64, z16):
    @functools.partial(
        pl.kernel,
        out_type=(
            jax.ShapeDtypeStruct((NC, NPAD, HHALF), jnp.float32),
            jax.ShapeDtypeStruct((NPAD, LANES), jnp.float32),
        ),
        mesh=_vector_mesh(),
        scratch_types=[
            pltpu.VMEM((NPAD * 4,), jnp.int32),
            pltpu.VMEM((N_REL * 4,), jnp.float32),
            pltpu.VMEM((EK,), jnp.int32),
            pltpu.VMEM((EK,), jnp.int32),
            pltpu.VMEM((EK,), jnp.int32),
            pltpu.VMEM((EK,), jnp.int32),
            pltpu.VMEM((EK, HHALF), jnp.float32),
            pltpu.VMEM((EK, LANES), jnp.float32),
            pltpu.VMEM_SHARED((NPAD, HHALF), jnp.float32),
            pltpu.VMEM_SHARED((NPAD, LANES), jnp.float32),
            pltpu.SemaphoreType.DMA,
        ],
        compiler_params=_sc_compiler_params(),
    )
    def k(src_hbm, dst_hbm, rel_hbm, whs_hbm, eler_hbm, ee_hbm, z64_hbm, z16_hbm,
          outp_hbm, den_hbm,
          eler_v, ee_v, src_v, dst_v, rel_v, srcw_v, rows_v, w_v,
          out_sh, den_sh, sem):
        cid = lax.axis_index("c")
        sid = lax.axis_index("s")
        r0 = sid * ROWS_PER_SUB
        is0 = cid == 0

        pltpu.sync_copy(z64_hbm.at[pl.ds(r0, ROWS_PER_SUB)],
                        out_sh.at[pl.ds(r0, ROWS_PER_SUB)])
        pltpu.sync_copy(z16_hbm.at[pl.ds(r0, ROWS_PER_SUB)],
                        den_sh.at[pl.ds(r0, ROWS_PER_SUB)])
        pltpu.sync_copy(eler_hbm, eler_v)
        pltpu.sync_copy(ee_hbm, ee_v)
        zero16 = jnp.zeros((LANES,), jnp.float32)

        @pl.loop(0, EK)
        def _(kk):
            w_v[kk, :] = zero16

        plsc.subcore_barrier()

        iota16 = lax.iota(jnp.int32, 16)
        whoff = cid * NPAD

        @pl.loop(0, N_CHUNKS)
        def _(g):
            base = sid * E_PER_S + g * EK
            pltpu.sync_copy(src_hbm.at[pl.ds(base, EK)], src_v)
            pltpu.sync_copy(dst_hbm.at[pl.ds(base, EK)], dst_v)
            pltpu.sync_copy(rel_hbm.at[pl.ds(base, EK)], rel_v)

            @pl.loop(0, EK // LANES)
            def _(q):
                sl = pl.ds(q * LANES, LANES)
                srcw_v[sl] = src_v[sl] + whoff

            gat = pltpu.async_copy(whs_hbm.at[srcw_v], rows_v, sem)

            @pl.loop(0, EK // LANES)
            def _(q):
                sl = pl.ds(q * LANES, LANES)
                s16 = src_v[sl]
                d16 = dst_v[sl]
                r16 = rel_v[sl]
                for hh in range(HEADS):
                    sw = plsc.load_gather(eler_v, [s16 * 4 + hh])
                    dw = plsc.load_gather(eler_v, [d16 * 4 + hh])
                    ee = plsc.load_gather(ee_v, [r16 * 4 + hh])
                    el = plsc.bitcast(sw & jnp.int32(-65536), jnp.float32)
                    er = plsc.bitcast(dw << 16, jnp.float32)
                    e = el + er + ee
                    e = jnp.maximum(e, e * NEG_SLOPE)
                    w = jnp.exp(e)
                    plsc.store_scatter(
                        w_v,
                        [q * LANES + iota16, jnp.full((LANES,), hh, jnp.int32)],
                        w)

            @pl.when(is0)
            def _():
                pltpu.sync_copy(w_v, den_sh.at[dst_v], add=True)

            gat.wait()

            @pl.loop(0, EK)
            def _(kk):
                w16 = w_v[kk, :]
                for cc in range(HHALF // LANES):
                    m0 = w16[cc // 2]
                    m1 = w16[2 + cc // 2]
                    m = jnp.full((LANES,), jnp.where(is0, m0, m1))
                    sl = pl.ds(cc * LANES, LANES)
                    rows_v[kk, sl] = rows_v[kk, sl] * m

            pltpu.sync_copy(rows_v, out_sh.at[dst_v], add=True)

        plsc.subcore_barrier()
        pltpu.sync_copy(out_sh.at[pl.ds(r0, ROWS_PER_SUB)],
                        outp_hbm.at[cid, pl.ds(r0, ROWS_PER_SUB)])

        @pl.when(is0)
        def _():
            pltpu.sync_copy(den_sh.at[pl.ds(r0, ROWS_PER_SUB)],
                            den_hbm.at[pl.ds(r0, ROWS_PER_SUB)])

    return k(src, dst, rel, whs_flat, eler_flat, ee_flat, z64, z16)


def _dot(a, b):
    return jnp.dot(a, b, preferred_element_type=jnp.float32)


def _pack_eler(eler):
    eb = lax.bitcast_convert_type(eler, jnp.int32)
    el_b = eb[:, :HEADS] & jnp.int32(-65536)
    er_b = lax.shift_right_logical(eb[:, HEADS:], 16)
    return el_b | er_b


def _tc_prep(h, W, wcat, rel_table, wrcat):
    def body(h_ref, w_ref, wcat_ref, rt_ref, wrcat_ref,
             whs_ref, eler_ref, ee_ref):
        hh = h_ref[...]
        whs_ref[0] = _dot(hh, w_ref[:, :HHALF])
        whs_ref[1] = _dot(hh, w_ref[:, HHALF:])
        eler_ref[...] = _pack_eler(_dot(hh, wcat_ref[...]))
        ee_ref[...] = _dot(rt_ref[...], wrcat_ref[...])

    return pl.pallas_call(
        body,
        out_shape=(
            jax.ShapeDtypeStruct((NC, NPAD, HHALF), jnp.float32),
            jax.ShapeDtypeStruct((NPAD, HEADS), jnp.int32),
            jax.ShapeDtypeStruct((N_REL, HEADS), jnp.float32),
        ),
    )(h, W, wcat, rel_table, wrcat)


def _combine_norm(o_ref, d_ref, exp_ref):
    s = jnp.concatenate([o_ref[0], o_ref[1]], axis=1)
    inv = 1.0 / (d_ref[:, :HEADS] + 1e-9)
    return s * _dot(inv, exp_ref[...])


def _tc_norm_prep(outp, den, W, wcat, rel_table, wrcat, expand):
    def body(o_ref, d_ref, w_ref, wcat_ref, rt_ref, wrcat_ref, exp_ref,
             whs_ref, eler_ref, ee_ref):
        h2 = _combine_norm(o_ref, d_ref, exp_ref)
        whs_ref[0] = _dot(h2, w_ref[:, :HHALF])
        whs_ref[1] = _dot(h2, w_ref[:, HHALF:])
        eler_ref[...] = _pack_eler(_dot(h2, wcat_ref[...]))
        ee_ref[...] = _dot(rt_ref[...], wrcat_ref[...])

    return pl.pallas_call(
        body,
        out_shape=(
            jax.ShapeDtypeStruct((NC, NPAD, HHALF), jnp.float32),
            jax.ShapeDtypeStruct((NPAD, HEADS), jnp.int32),
            jax.ShapeDtypeStruct((N_REL, HEADS), jnp.float32),
        ),
    )(outp, den, W, wcat, rel_table, wrcat, expand)


def _tc_final(outp, den, cls_idx, expand):
    n_cls = cls_idx.shape[0]

    def body(o_ref, d_ref, cls_ref, exp_ref, out_ref):
        h3 = _combine_norm(o_ref, d_ref, exp_ref)
        ids = cls_ref[...]
        col = lax.broadcasted_iota(jnp.int32, (n_cls, NPAD), 1)
        onehot = (ids[:, None] == col).astype(jnp.float32)
        out_ref[...] = _dot(onehot, h3)

    return pl.pallas_call(
        body,
        out_shape=jax.ShapeDtypeStruct((n_cls, HID), jnp.float32),
    )(outp, den, cls_idx, expand)


def _fold_attn(W, a_src, a_dst):
    Wr3 = W.reshape(W.shape[0], HEADS, D_HEAD)
    vsrc = jnp.einsum("khd,hd->kh", Wr3, a_src)
    vdst = jnp.einsum("khd,hd->kh", Wr3, a_dst)
    return jnp.concatenate([vsrc, vdst], axis=1)


def _fold_rel(Wr, a_rel):
    return jnp.einsum("khd,hd->kh", Wr.reshape(Wr.shape[0], HEADS, D_HEAD), a_rel)


_EXPAND = np.zeros((HEADS, HID), np.float32)
for _h in range(HEADS):
    _EXPAND[_h, _h * D_HEAD:(_h + 1) * D_HEAD] = 1.0


def kernel(ent_table, rel_table, W1, Wr1, a_src1, a_dst1, a_rel1,
           W2, Wr2, a_src2, a_dst2, a_rel2,
           ent_ids, rel_ids, edge_index, cls_idx):
    expand = jnp.asarray(_EXPAND)
    wcat1 = _fold_attn(W1, a_src1, a_dst1)
    wcat2 = _fold_attn(W2, a_src2, a_dst2)
    wrcat1 = _fold_rel(Wr1, a_rel1)
    wrcat2 = _fold_rel(Wr2, a_rel2)

    src = edge_index[0]
    dst = edge_index[1]
    z64 = jnp.zeros((NPAD, HHALF), jnp.float32)
    z16 = jnp.zeros((NPAD, LANES), jnp.float32)

    ids_pad = jnp.pad(ent_ids, (0, NPAD - N_NODES))
    h = _sc_gather_rows(ent_table, ids_pad)

    whs1, eler1, ee1 = _tc_prep(h, W1, wcat1, rel_table, wrcat1)
    outp1, den1 = _sc_edge_pass(src, dst, rel_ids, whs1.reshape(NC * NPAD, HHALF),
                                eler1.reshape(-1), ee1.reshape(-1), z64, z16)

    whs2, eler2, ee2 = _tc_norm_prep(outp1, den1, W2, wcat2, rel_table, wrcat2,
                                     expand)
    outp2, den2 = _sc_edge_pass(src, dst, rel_ids, whs2.reshape(NC * NPAD, HHALF),
                                eler2.reshape(-1), ee2.reshape(-1), z64, z16)

    return _tc_final(outp2, den2, cls_idx, expand)


# speedup vs baseline: 45.9435x; 1.6221x over previous
import dataclasses
import functools

import numpy as np
import jax
import jax.numpy as jnp
from jax import lax
from jax.experimental import pallas as pl
from jax.experimental.pallas import tpu as pltpu
from jax.experimental.pallas import tpu_sc as plsc

N_NODES = 10000
N_EDGES = 320000
N_REL = 1000
HID = 128
HEADS = 4
D_HEAD = HID // HEADS
NEG_SLOPE = 0.2

NC = 2
NS = 16
LANES = 16
NW = NC * NS

HHALF = HID // NC
EK = 80
E_PER_S = N_EDGES // NS
N_CHUNKS = E_PER_S // EK

NPAD = 10240
GPW = NPAD // NW
ROWS_PER_SUB = NPAD // NS


def _vector_mesh():
    return plsc.VectorSubcoreMesh(core_axis_name="c", subcore_axis_name="s")


def _sc_compiler_params():
    cp = pltpu.CompilerParams()
    fields = pltpu.CompilerParams.__dataclass_fields__
    if "needs_layout_passes" in fields:
        cp = dataclasses.replace(cp, needs_layout_passes=False)
    if "use_tc_tiling_on_sc" in fields:
        cp = dataclasses.replace(cp, use_tc_tiling_on_sc=False)
    return cp


@jax.jit
def _sc_gather_rows(table, idx):
    n_sub = GPW // EK

    @functools.partial(
        pl.kernel,
        out_type=jax.ShapeDtypeStruct((NPAD, HID), jnp.float32),
        mesh=_vector_mesh(),
        scratch_types=[
            pltpu.VMEM((EK,), jnp.int32),
            pltpu.VMEM((EK, HID), jnp.float32),
            pltpu.SemaphoreType.DMA,
        ],
    )
    def k(table_hbm, idx_hbm, out_hbm, idx_v, rows_v, sem):
        wid = lax.axis_index("s") * NC + lax.axis_index("c")

        @pl.loop(0, n_sub)
        def _(g):
            base = wid * GPW + g * EK
            pltpu.sync_copy(idx_hbm.at[pl.ds(base, EK)], idx_v)
            pltpu.async_copy(table_hbm.at[idx_v], rows_v, sem).wait()
            pltpu.sync_copy(rows_v, out_hbm.at[pl.ds(base, EK)])

    return k(table, idx)


IDS_BLK = 3 * EK


@jax.jit
def _sc_edge_pass(ids_packed, whs_flat, eler_flat, ee_flat, z64, z16):
    @functools.partial(
        pl.kernel,
        out_type=(
            jax.ShapeDtypeStruct((NC, NPAD, HHALF), jnp.float32),
            jax.ShapeDtypeStruct((NPAD, LANES), jnp.float32),
        ),
        mesh=_vector_mesh(),
        scratch_types=[
            pltpu.VMEM((NPAD * 4,), jnp.int32),
            pltpu.VMEM((N_REL * 4,), jnp.float32),
            pltpu.VMEM((IDS_BLK,), jnp.int32),
            pltpu.VMEM((IDS_BLK,), jnp.int32),
            pltpu.VMEM((EK,), jnp.int32),
            pltpu.VMEM((EK,), jnp.int32),
            pltpu.VMEM((EK,), jnp.int32),
            pltpu.VMEM((EK,), jnp.int32),
            pltpu.VMEM((EK, HHALF), jnp.float32),
            pltpu.VMEM((EK, HHALF), jnp.float32),
            pltpu.VMEM((EK, LANES), jnp.float32),
            pltpu.VMEM((EK, LANES), jnp.float32),
            pltpu.VMEM_SHARED((NPAD, HHALF), jnp.float32),
            pltpu.VMEM_SHARED((NPAD, LANES), jnp.float32),
            pltpu.SemaphoreType.DMA,
            pltpu.SemaphoreType.DMA,
            pltpu.SemaphoreType.DMA,
            pltpu.SemaphoreType.DMA,
            pltpu.SemaphoreType.DMA,
            pltpu.SemaphoreType.DMA,
            pltpu.SemaphoreType.DMA,
            pltpu.SemaphoreType.DMA,
        ],
        compiler_params=_sc_compiler_params(),
    )
    def k(ids_hbm, whs_hbm, eler_hbm, ee_hbm, z64_hbm, z16_hbm,
          outp_hbm, den_hbm,
          eler_v, ee_v, ids0, ids1, srcw0, srcw1, dsti0, dsti1,
          rows0, rows1, w0, w1, out_sh, den_sh,
          si0, si1, sg0, sg1, so0, so1, sd0, sd1):
        cid = lax.axis_index("c")
        sid = lax.axis_index("s")
        r0 = sid * ROWS_PER_SUB
        is0 = cid == 0

        pltpu.sync_copy(z64_hbm.at[pl.ds(r0, ROWS_PER_SUB)],
                        out_sh.at[pl.ds(r0, ROWS_PER_SUB)])
        pltpu.sync_copy(z16_hbm.at[pl.ds(r0, ROWS_PER_SUB)],
                        den_sh.at[pl.ds(r0, ROWS_PER_SUB)])
        pltpu.sync_copy(eler_hbm, eler_v)
        pltpu.sync_copy(ee_hbm, ee_v)
        zero16 = jnp.zeros((LANES,), jnp.float32)

        @pl.loop(0, EK)
        def _(kk):
            w0[kk, :] = zero16
            w1[kk, :] = zero16

        plsc.subcore_barrier()

        iota16 = lax.iota(jnp.int32, 16)
        whoff = cid * NPAD
        tile_base = sid * N_CHUNKS * IDS_BLK

        bufs = ((ids0, srcw0, dsti0, rows0, w0, si0, sg0, so0, sd0),
                (ids1, srcw1, dsti1, rows1, w1, si1, sg1, so1, sd1))

        pltpu.async_copy(ids_hbm.at[pl.ds(tile_base, IDS_BLK)], ids0, si0)

        @pl.loop(0, N_CHUNKS // 2)
        def _(j):
            for p in (0, 1):
                ids_v, srcw_v, dsti_v, rows_v, w_v, si, sg, so, sd = bufs[p]
                n_ids, n_si = bufs[1 - p][0], bufs[1 - p][5]
                g = j * 2 + p
                pltpu.make_async_copy(
                    ids_hbm.at[pl.ds(0, IDS_BLK)], ids_v, si).wait()
                pltpu.async_copy(
                    ids_hbm.at[pl.ds(tile_base + (g + 1) * IDS_BLK, IDS_BLK)],
                    n_ids, n_si)
                @pl.when(j > 0)
                def _():
                    pltpu.make_async_copy(
                        rows_v, out_sh.at[dsti_v], so).wait()

                    @pl.when(is0)
                    def _():
                        pltpu.make_async_copy(
                            w_v, den_sh.at[dsti_v], sd).wait()

                @pl.loop(0, EK // LANES)
                def _(q):
                    sl = pl.ds(q * LANES, LANES)
                    srcw_v[sl] = ids_v[pl.ds(q * LANES, LANES)] + whoff
                    dsti_v[sl] = ids_v[pl.ds(EK + q * LANES, LANES)]

                gat = pltpu.async_copy(whs_hbm.at[srcw_v], rows_v, sg)

                @pl.loop(0, EK // LANES)
                def _(q):
                    s16 = ids_v[pl.ds(q * LANES, LANES)]
                    d16 = ids_v[pl.ds(EK + q * LANES, LANES)]
                    r16 = ids_v[pl.ds(2 * EK + q * LANES, LANES)]
                    for hh in range(HEADS):
                        sw = plsc.load_gather(eler_v, [s16 * 4 + hh])
                        dw = plsc.load_gather(eler_v, [d16 * 4 + hh])
                        ee = plsc.load_gather(ee_v, [r16 * 4 + hh])
                        el = plsc.bitcast(sw & jnp.int32(-65536), jnp.float32)
                        er = plsc.bitcast(dw << 16, jnp.float32)
                        e = el + er + ee
                        e = jnp.maximum(e, e * NEG_SLOPE)
                        w = jnp.exp(e)
                        plsc.store_scatter(
                            w_v,
                            [q * LANES + iota16,
                             jnp.full((LANES,), hh, jnp.int32)],
                            w)

                @pl.when(is0)
                def _():
                    pltpu.async_copy(w_v, den_sh.at[dsti_v], sd, add=True)

                gat.wait()

                @pl.loop(0, EK)
                def _(kk):
                    w16 = w_v[kk, :]
                    for cc in range(HHALF // LANES):
                        m0 = w16[cc // 2]
                        m1 = w16[2 + cc // 2]
                        m = jnp.full((LANES,), jnp.where(is0, m0, m1))
                        sl = pl.ds(cc * LANES, LANES)
                        rows_v[kk, sl] = rows_v[kk, sl] * m

                pltpu.async_copy(rows_v, out_sh.at[dsti_v], so, add=True)

        pltpu.make_async_copy(ids_hbm.at[pl.ds(0, IDS_BLK)], ids0, si0).wait()
        for p in (0, 1):
            ids_v, srcw_v, dsti_v, rows_v, w_v, si, sg, so, sd = bufs[p]
            pltpu.make_async_copy(rows_v, out_sh.at[dsti_v], so).wait()

            @pl.when(is0)
            def _():
                pltpu.make_async_copy(w_v, den_sh.at[dsti_v], sd).wait()

        plsc.subcore_barrier()
        pltpu.sync_copy(out_sh.at[pl.ds(r0, ROWS_PER_SUB)],
                        outp_hbm.at[cid, pl.ds(r0, ROWS_PER_SUB)])

        @pl.when(is0)
        def _():
            pltpu.sync_copy(den_sh.at[pl.ds(r0, ROWS_PER_SUB)],
                            den_hbm.at[pl.ds(r0, ROWS_PER_SUB)])

    return k(ids_packed, whs_flat, eler_flat, ee_flat, z64, z16)


def _dot(a, b):
    return jnp.dot(a, b, preferred_element_type=jnp.float32)


def _pack_eler(eler):
    eb = lax.bitcast_convert_type(eler, jnp.int32)
    el_b = eb[:, :HEADS] & jnp.int32(-65536)
    er_b = lax.shift_right_logical(eb[:, HEADS:], 16)
    return el_b | er_b


def _tc_prep(h, W, wcat, rel_table, wrcat):
    def body(h_ref, w_ref, wcat_ref, rt_ref, wrcat_ref,
             whs_ref, eler_ref, ee_ref):
        hh = h_ref[...]
        whs_ref[0] = _dot(hh, w_ref[:, :HHALF])
        whs_ref[1] = _dot(hh, w_ref[:, HHALF:])
        eler_ref[...] = _pack_eler(_dot(hh, wcat_ref[...]))
        ee_ref[...] = _dot(rt_ref[...], wrcat_ref[...])

    return pl.pallas_call(
        body,
        out_shape=(
            jax.ShapeDtypeStruct((NC, NPAD, HHALF), jnp.float32),
            jax.ShapeDtypeStruct((NPAD, HEADS), jnp.int32),
            jax.ShapeDtypeStruct((N_REL, HEADS), jnp.float32),
        ),
    )(h, W, wcat, rel_table, wrcat)


def _combine_norm(o_ref, d_ref, exp_ref):
    s = jnp.concatenate([o_ref[0], o_ref[1]], axis=1)
    inv = 1.0 / (d_ref[:, :HEADS] + 1e-9)
    return s * _dot(inv, exp_ref[...])


def _tc_norm_prep(outp, den, W, wcat, rel_table, wrcat, expand):
    def body(o_ref, d_ref, w_ref, wcat_ref, rt_ref, wrcat_ref, exp_ref,
             whs_ref, eler_ref, ee_ref):
        h2 = _combine_norm(o_ref, d_ref, exp_ref)
        whs_ref[0] = _dot(h2, w_ref[:, :HHALF])
        whs_ref[1] = _dot(h2, w_ref[:, HHALF:])
        eler_ref[...] = _pack_eler(_dot(h2, wcat_ref[...]))
        ee_ref[...] = _dot(rt_ref[...], wrcat_ref[...])

    return pl.pallas_call(
        body,
        out_shape=(
            jax.ShapeDtypeStruct((NC, NPAD, HHALF), jnp.float32),
            jax.ShapeDtypeStruct((NPAD, HEADS), jnp.int32),
            jax.ShapeDtypeStruct((N_REL, HEADS), jnp.float32),
        ),
    )(outp, den, W, wcat, rel_table, wrcat, expand)


def _tc_final(outp, den, cls_idx, expand):
    n_cls = cls_idx.shape[0]

    def body(o_ref, d_ref, cls_ref, exp_ref, out_ref):
        h3 = _combine_norm(o_ref, d_ref, exp_ref)
        ids = cls_ref[...]
        col = lax.broadcasted_iota(jnp.int32, (n_cls, NPAD), 1)
        onehot = (ids[:, None] == col).astype(jnp.float32)
        out_ref[...] = _dot(onehot, h3)

    return pl.pallas_call(
        body,
        out_shape=jax.ShapeDtypeStruct((n_cls, HID), jnp.float32),
    )(outp, den, cls_idx, expand)


def _fold_attn(W, a_src, a_dst):
    Wr3 = W.reshape(W.shape[0], HEADS, D_HEAD)
    vsrc = jnp.einsum("khd,hd->kh", Wr3, a_src)
    vdst = jnp.einsum("khd,hd->kh", Wr3, a_dst)
    return jnp.concatenate([vsrc, vdst], axis=1)


def _fold_rel(Wr, a_rel):
    return jnp.einsum("khd,hd->kh", Wr.reshape(Wr.shape[0], HEADS, D_HEAD), a_rel)


_EXPAND = np.zeros((HEADS, HID), np.float32)
for _h in range(HEADS):
    _EXPAND[_h, _h * D_HEAD:(_h + 1) * D_HEAD] = 1.0


def kernel(ent_table, rel_table, W1, Wr1, a_src1, a_dst1, a_rel1,
           W2, Wr2, a_src2, a_dst2, a_rel2,
           ent_ids, rel_ids, edge_index, cls_idx):
    expand = jnp.asarray(_EXPAND)
    wcat1 = _fold_attn(W1, a_src1, a_dst1)
    wcat2 = _fold_attn(W2, a_src2, a_dst2)
    wrcat1 = _fold_rel(Wr1, a_rel1)
    wrcat2 = _fold_rel(Wr2, a_rel2)

    ids = jnp.stack([edge_index[0], edge_index[1], rel_ids])
    ids = ids.reshape(3, NS, N_CHUNKS, EK).transpose(1, 2, 0, 3).reshape(-1)
    ids = jnp.concatenate([ids, jnp.zeros((IDS_BLK,), jnp.int32)])
    z64 = jnp.zeros((NPAD, HHALF), jnp.float32)
    z16 = jnp.zeros((NPAD, LANES), jnp.float32)

    ids_pad = jnp.pad(ent_ids, (0, NPAD - N_NODES))
    h = _sc_gather_rows(ent_table, ids_pad)

    whs1, eler1, ee1 = _tc_prep(h, W1, wcat1, rel_table, wrcat1)
    outp1, den1 = _sc_edge_pass(ids, whs1.reshape(NC * NPAD, HHALF),
                                eler1.reshape(-1), ee1.reshape(-1), z64, z16)

    whs2, eler2, ee2 = _tc_norm_prep(outp1, den1, W2, wcat2, rel_table, wrcat2,
                                     expand)
    outp2, den2 = _sc_edge_pass(ids, whs2.reshape(NC * NPAD, HHALF),
                                eler2.reshape(-1), ee2.reshape(-1), z64, z16)

    return _tc_final(outp2, den2, cls_idx, expand)


# speedup vs baseline: 64.3348x; 1.4003x over previous
import dataclasses
import functools

import numpy as np
import jax
import jax.numpy as jnp
from jax import lax
from jax.experimental import pallas as pl
from jax.experimental.pallas import tpu as pltpu
from jax.experimental.pallas import tpu_sc as plsc

N_NODES = 10000
N_EDGES = 320000
N_REL = 1000
HID = 128
HEADS = 4
D_HEAD = HID // HEADS
NEG_SLOPE = 0.2

NC = 2
NS = 16
LANES = 16
NW = NC * NS

HHALF = HID // NC
EK = 80
E_PER_S = N_EDGES // NS
N_CHUNKS = E_PER_S // EK

NPAD = 10240
GPW = NPAD // NW
ROWS_PER_SUB = NPAD // NS


def _vector_mesh():
    return plsc.VectorSubcoreMesh(core_axis_name="c", subcore_axis_name="s")


def _sc_compiler_params():
    cp = pltpu.CompilerParams()
    fields = pltpu.CompilerParams.__dataclass_fields__
    if "needs_layout_passes" in fields:
        cp = dataclasses.replace(cp, needs_layout_passes=False)
    if "use_tc_tiling_on_sc" in fields:
        cp = dataclasses.replace(cp, use_tc_tiling_on_sc=False)
    return cp


@jax.jit
def _sc_gather_rows(table, idx):
    n_sub = GPW // EK

    @functools.partial(
        pl.kernel,
        out_type=jax.ShapeDtypeStruct((NPAD, HID), jnp.float32),
        mesh=_vector_mesh(),
        scratch_types=[
            pltpu.VMEM((EK,), jnp.int32),
            pltpu.VMEM((EK, HID), jnp.float32),
            pltpu.SemaphoreType.DMA,
        ],
    )
    def k(table_hbm, idx_hbm, out_hbm, idx_v, rows_v, sem):
        wid = lax.axis_index("s") * NC + lax.axis_index("c")

        @pl.loop(0, n_sub)
        def _(g):
            base = wid * GPW + g * EK
            pltpu.sync_copy(idx_hbm.at[pl.ds(base, EK)], idx_v)
            pltpu.async_copy(table_hbm.at[idx_v], rows_v, sem).wait()
            pltpu.sync_copy(rows_v, out_hbm.at[pl.ds(base, EK)])

    return k(table, idx)


IDS_BLK = 3 * EK


@jax.jit
def _sc_edge_pass(ids_packed, whs_flat, eler_flat, ee_flat, z64, z16):
    @functools.partial(
        pl.kernel,
        out_type=(
            jax.ShapeDtypeStruct((NC, NPAD, HHALF), jnp.float32),
            jax.ShapeDtypeStruct((NC, NPAD, LANES), jnp.float32),
        ),
        mesh=_vector_mesh(),
        scratch_types=[
            pltpu.VMEM((NPAD * 4,), jnp.int32),
            pltpu.VMEM((N_REL * 4,), jnp.float32),
            pltpu.VMEM((IDS_BLK,), jnp.int32),
            pltpu.VMEM((IDS_BLK,), jnp.int32),
            pltpu.VMEM((EK,), jnp.int32),
            pltpu.VMEM((EK,), jnp.int32),
            pltpu.VMEM((EK,), jnp.int32),
            pltpu.VMEM((EK,), jnp.int32),
            pltpu.VMEM((EK, HHALF), jnp.float32),
            pltpu.VMEM((EK, HHALF), jnp.float32),
            pltpu.VMEM((EK, LANES), jnp.float32),
            pltpu.VMEM((EK, LANES), jnp.float32),
            pltpu.VMEM_SHARED((NPAD, HHALF), jnp.float32),
            pltpu.VMEM_SHARED((NPAD, LANES), jnp.float32),
            pltpu.SemaphoreType.DMA,
            pltpu.SemaphoreType.DMA,
            pltpu.SemaphoreType.DMA,
            pltpu.SemaphoreType.DMA,
            pltpu.SemaphoreType.DMA,
            pltpu.SemaphoreType.DMA,
            pltpu.SemaphoreType.DMA,
            pltpu.SemaphoreType.DMA,
        ],
        compiler_params=_sc_compiler_params(),
    )
    def k(ids_hbm, whs_hbm, eler_hbm, ee_hbm, z64_hbm, z16_hbm,
          outp_hbm, den_hbm,
          eler_v, ee_v, ids0, ids1, srcw0, srcw1, dsti0, dsti1,
          rows0, rows1, w0, w1, out_sh, den_sh,
          si0, si1, sg0, sg1, so0, so1, sd0, sd1):
        cid = lax.axis_index("c")
        sid = lax.axis_index("s")
        r0 = sid * ROWS_PER_SUB
        is0 = cid == 0

        pltpu.sync_copy(z64_hbm.at[pl.ds(r0, ROWS_PER_SUB)],
                        out_sh.at[pl.ds(r0, ROWS_PER_SUB)])
        pltpu.sync_copy(z16_hbm.at[pl.ds(r0, ROWS_PER_SUB)],
                        den_sh.at[pl.ds(r0, ROWS_PER_SUB)])
        pltpu.sync_copy(eler_hbm, eler_v)
        pltpu.sync_copy(ee_hbm, ee_v)
        zero16 = jnp.zeros((LANES,), jnp.float32)

        @pl.loop(0, EK)
        def _(kk):
            w0[kk, :] = zero16
            w1[kk, :] = zero16

        plsc.subcore_barrier()

        iota16 = lax.iota(jnp.int32, 16)
        hbase = cid * 2
        whoff = cid * NPAD
        tile_base = sid * N_CHUNKS * IDS_BLK

        bufs = ((ids0, srcw0, dsti0, rows0, w0, si0, sg0, so0, sd0),
                (ids1, srcw1, dsti1, rows1, w1, si1, sg1, so1, sd1))

        pltpu.async_copy(ids_hbm.at[pl.ds(tile_base, IDS_BLK)], ids0, si0)

        @pl.loop(0, N_CHUNKS // 2)
        def _(j):
            for p in (0, 1):
                ids_v, srcw_v, dsti_v, rows_v, w_v, si, sg, so, sd = bufs[p]
                n_ids, n_si = bufs[1 - p][0], bufs[1 - p][5]
                g = j * 2 + p
                pltpu.make_async_copy(
                    ids_hbm.at[pl.ds(0, IDS_BLK)], ids_v, si).wait()
                pltpu.async_copy(
                    ids_hbm.at[pl.ds(tile_base + (g + 1) * IDS_BLK, IDS_BLK)],
                    n_ids, n_si)
                @pl.when(j > 0)
                def _():
                    pltpu.make_async_copy(
                        rows_v, out_sh.at[dsti_v], so).wait()
                    pltpu.make_async_copy(
                        w_v, den_sh.at[dsti_v], sd).wait()

                @pl.loop(0, EK // LANES)
                def _(q):
                    sl = pl.ds(q * LANES, LANES)
                    srcw_v[sl] = ids_v[pl.ds(q * LANES, LANES)] + whoff
                    dsti_v[sl] = ids_v[pl.ds(EK + q * LANES, LANES)]

                gat = pltpu.async_copy(whs_hbm.at[srcw_v], rows_v, sg)

                @pl.loop(0, EK // LANES)
                def _(q):
                    s16 = ids_v[pl.ds(q * LANES, LANES)] * 4 + hbase
                    d16 = ids_v[pl.ds(EK + q * LANES, LANES)] * 4 + hbase
                    r16 = ids_v[pl.ds(2 * EK + q * LANES, LANES)] * 4 + hbase
                    for hh in range(2):
                        sw = plsc.load_gather(eler_v, [s16 + hh])
                        dw = plsc.load_gather(eler_v, [d16 + hh])
                        ee = plsc.load_gather(ee_v, [r16 + hh])
                        el = plsc.bitcast(sw & jnp.int32(-65536), jnp.float32)
                        er = plsc.bitcast(dw << 16, jnp.float32)
                        e = el + er + ee
                        e = jnp.maximum(e, e * NEG_SLOPE)
                        w = jnp.exp(e)
                        plsc.store_scatter(
                            w_v,
                            [q * LANES + iota16,
                             jnp.full((LANES,), hh, jnp.int32)],
                            w)

                pltpu.async_copy(w_v, den_sh.at[dsti_v], sd, add=True)

                gat.wait()

                @pl.loop(0, EK)
                def _(kk):
                    w16 = w_v[kk, :]
                    for cc in range(HHALF // LANES):
                        m = jnp.full((LANES,), w16[cc // 2])
                        sl = pl.ds(cc * LANES, LANES)
                        rows_v[kk, sl] = rows_v[kk, sl] * m

                pltpu.async_copy(rows_v, out_sh.at[dsti_v], so, add=True)

        pltpu.make_async_copy(ids_hbm.at[pl.ds(0, IDS_BLK)], ids0, si0).wait()
        for p in (0, 1):
            ids_v, srcw_v, dsti_v, rows_v, w_v, si, sg, so, sd = bufs[p]
            pltpu.make_async_copy(rows_v, out_sh.at[dsti_v], so).wait()
            pltpu.make_async_copy(w_v, den_sh.at[dsti_v], sd).wait()

        plsc.subcore_barrier()
        pltpu.sync_copy(out_sh.at[pl.ds(r0, ROWS_PER_SUB)],
                        outp_hbm.at[cid, pl.ds(r0, ROWS_PER_SUB)])
        pltpu.sync_copy(den_sh.at[pl.ds(r0, ROWS_PER_SUB)],
                        den_hbm.at[cid, pl.ds(r0, ROWS_PER_SUB)])

    return k(ids_packed, whs_flat, eler_flat, ee_flat, z64, z16)


def _dot(a, b):
    return jnp.dot(a, b, preferred_element_type=jnp.float32)


def _pack_eler(eler):
    eb = lax.bitcast_convert_type(eler, jnp.int32)
    el_b = eb[:, :HEADS] & jnp.int32(-65536)
    er_b = lax.shift_right_logical(eb[:, HEADS:], 16)
    return el_b | er_b


def _tc_prep(h, W, wcat, rel_table, wrcat):
    def body(h_ref, w_ref, wcat_ref, rt_ref, wrcat_ref,
             whs_ref, eler_ref, ee_ref):
        hh = h_ref[...]
        whs_ref[0] = _dot(hh, w_ref[:, :HHALF])
        whs_ref[1] = _dot(hh, w_ref[:, HHALF:])
        eler_ref[...] = _pack_eler(_dot(hh, wcat_ref[...]))
        ee_ref[...] = _dot(rt_ref[...], wrcat_ref[...])

    return pl.pallas_call(
        body,
        out_shape=(
            jax.ShapeDtypeStruct((NC, NPAD, HHALF), jnp.float32),
            jax.ShapeDtypeStruct((NPAD, HEADS), jnp.int32),
            jax.ShapeDtypeStruct((N_REL, HEADS), jnp.float32),
        ),
    )(h, W, wcat, rel_table, wrcat)


def _combine_norm(o_ref, d_ref, exp_ref):
    s = jnp.concatenate([o_ref[0], o_ref[1]], axis=1)
    den4 = jnp.concatenate([d_ref[0][:, :2], d_ref[1][:, :2]], axis=1)
    inv = 1.0 / (den4 + 1e-9)
    return s * _dot(inv, exp_ref[...])


def _tc_norm_prep(outp, den, W, wcat, rel_table, wrcat, expand):
    def body(o_ref, d_ref, w_ref, wcat_ref, rt_ref, wrcat_ref, exp_ref,
             whs_ref, eler_ref, ee_ref):
        h2 = _combine_norm(o_ref, d_ref, exp_ref)
        whs_ref[0] = _dot(h2, w_ref[:, :HHALF])
        whs_ref[1] = _dot(h2, w_ref[:, HHALF:])
        eler_ref[...] = _pack_eler(_dot(h2, wcat_ref[...]))
        ee_ref[...] = _dot(rt_ref[...], wrcat_ref[...])

    return pl.pallas_call(
        body,
        out_shape=(
            jax.ShapeDtypeStruct((NC, NPAD, HHALF), jnp.float32),
            jax.ShapeDtypeStruct((NPAD, HEADS), jnp.int32),
            jax.ShapeDtypeStruct((N_REL, HEADS), jnp.float32),
        ),
    )(outp, den, W, wcat, rel_table, wrcat, expand)


def _tc_final(outp, den, cls_idx, expand):
    n_cls = cls_idx.shape[0]

    def body(o_ref, d_ref, cls_ref, exp_ref, out_ref):
        h3 = _combine_norm(o_ref, d_ref, exp_ref)
        ids = cls_ref[...]
        col = lax.broadcasted_iota(jnp.int32, (n_cls, NPAD), 1)
        onehot = (ids[:, None] == col).astype(jnp.float32)
        out_ref[...] = _dot(onehot, h3)

    return pl.pallas_call(
        body,
        out_shape=jax.ShapeDtypeStruct((n_cls, HID), jnp.float32),
    )(outp, den, cls_idx, expand)


def _fold_attn(W, a_src, a_dst):
    Wr3 = W.reshape(W.shape[0], HEADS, D_HEAD)
    vsrc = jnp.einsum("khd,hd->kh", Wr3, a_src)
    vdst = jnp.einsum("khd,hd->kh", Wr3, a_dst)
    return jnp.concatenate([vsrc, vdst], axis=1)


def _fold_rel(Wr, a_rel):
    return jnp.einsum("khd,hd->kh", Wr.reshape(Wr.shape[0], HEADS, D_HEAD), a_rel)


_EXPAND = np.zeros((HEADS, HID), np.float32)
for _h in range(HEADS):
    _EXPAND[_h, _h * D_HEAD:(_h + 1) * D_HEAD] = 1.0


def kernel(ent_table, rel_table, W1, Wr1, a_src1, a_dst1, a_rel1,
           W2, Wr2, a_src2, a_dst2, a_rel2,
           ent_ids, rel_ids, edge_index, cls_idx):
    expand = jnp.asarray(_EXPAND)
    wcat1 = _fold_attn(W1, a_src1, a_dst1)
    wcat2 = _fold_attn(W2, a_src2, a_dst2)
    wrcat1 = _fold_rel(Wr1, a_rel1)
    wrcat2 = _fold_rel(Wr2, a_rel2)

    ids = jnp.stack([edge_index[0], edge_index[1], rel_ids])
    ids = ids.reshape(3, NS, N_CHUNKS, EK).transpose(1, 2, 0, 3).reshape(-1)
    ids = jnp.concatenate([ids, jnp.zeros((IDS_BLK,), jnp.int32)])
    z64 = jnp.zeros((NPAD, HHALF), jnp.float32)
    z16 = jnp.zeros((NPAD, LANES), jnp.float32)

    ids_pad = jnp.pad(ent_ids, (0, NPAD - N_NODES))
    h = _sc_gather_rows(ent_table, ids_pad)

    whs1, eler1, ee1 = _tc_prep(h, W1, wcat1, rel_table, wrcat1)
    outp1, den1 = _sc_edge_pass(ids, whs1.reshape(NC * NPAD, HHALF),
                                eler1.reshape(-1), ee1.reshape(-1), z64, z16)

    whs2, eler2, ee2 = _tc_norm_prep(outp1, den1, W2, wcat2, rel_table, wrcat2,
                                     expand)
    outp2, den2 = _sc_edge_pass(ids, whs2.reshape(NC * NPAD, HHALF),
                                eler2.reshape(-1), ee2.reshape(-1), z64, z16)

    return _tc_final(outp2, den2, cls_idx, expand)


# speedup vs baseline: 80.9703x; 1.2586x over previous
import dataclasses
import functools

import numpy as np
import jax
import jax.numpy as jnp
from jax import lax
from jax.experimental import pallas as pl
from jax.experimental.pallas import tpu as pltpu
from jax.experimental.pallas import tpu_sc as plsc

N_NODES = 10000
N_EDGES = 320000
N_REL = 1000
HID = 128
HEADS = 4
D_HEAD = HID // HEADS
NEG_SLOPE = 0.2

NC = 2
NS = 16
LANES = 16
NW = NC * NS

HHALF = HID // NC
EK = 80
E_PER_S = N_EDGES // NS
N_CHUNKS = E_PER_S // EK

NPAD = 10240
GPW = NPAD // NW
ROWS_PER_SUB = NPAD // NS


def _vector_mesh():
    return plsc.VectorSubcoreMesh(core_axis_name="c", subcore_axis_name="s")


def _sc_compiler_params():
    cp = pltpu.CompilerParams()
    fields = pltpu.CompilerParams.__dataclass_fields__
    if "needs_layout_passes" in fields:
        cp = dataclasses.replace(cp, needs_layout_passes=False)
    if "use_tc_tiling_on_sc" in fields:
        cp = dataclasses.replace(cp, use_tc_tiling_on_sc=False)
    return cp


@jax.jit
def _sc_gather_rows(table, idx):
    n_sub = GPW // EK

    @functools.partial(
        pl.kernel,
        out_type=jax.ShapeDtypeStruct((NPAD, HID), jnp.float32),
        mesh=_vector_mesh(),
        scratch_types=[
            pltpu.VMEM((EK,), jnp.int32),
            pltpu.VMEM((EK, HID), jnp.float32),
            pltpu.SemaphoreType.DMA,
        ],
    )
    def k(table_hbm, idx_hbm, out_hbm, idx_v, rows_v, sem):
        wid = lax.axis_index("s") * NC + lax.axis_index("c")

        @pl.loop(0, n_sub)
        def _(g):
            base = wid * GPW + g * EK
            pltpu.sync_copy(idx_hbm.at[pl.ds(base, EK)], idx_v)
            pltpu.async_copy(table_hbm.at[idx_v], rows_v, sem).wait()
            pltpu.sync_copy(rows_v, out_hbm.at[pl.ds(base, EK)])

    return k(table, idx)


IDS_BLK = 3 * EK


@jax.jit
def _sc_edge_pass(ids_packed, whs_flat, eler_flat, ee_flat, z64, z16):
    @functools.partial(
        pl.kernel,
        out_type=(
            jax.ShapeDtypeStruct((NC, NPAD, HHALF), jnp.float32),
            jax.ShapeDtypeStruct((NC, NPAD, LANES), jnp.float32),
        ),
        mesh=_vector_mesh(),
        scratch_types=[
            pltpu.VMEM((NPAD * 4,), jnp.int32),
            pltpu.VMEM((N_REL * 4,), jnp.float32),
            pltpu.VMEM((IDS_BLK,), jnp.int32),
            pltpu.VMEM((IDS_BLK,), jnp.int32),
            pltpu.VMEM((EK,), jnp.int32),
            pltpu.VMEM((EK,), jnp.int32),
            pltpu.VMEM((EK,), jnp.int32),
            pltpu.VMEM((EK,), jnp.int32),
            pltpu.VMEM((EK, HHALF), jnp.float32),
            pltpu.VMEM((EK, HHALF), jnp.float32),
            pltpu.VMEM((EK, LANES), jnp.float32),
            pltpu.VMEM((EK, LANES), jnp.float32),
            pltpu.VMEM_SHARED((NPAD, HHALF), jnp.float32),
            pltpu.VMEM_SHARED((NPAD, LANES), jnp.float32),
            pltpu.SemaphoreType.DMA,
            pltpu.SemaphoreType.DMA,
            pltpu.SemaphoreType.DMA,
            pltpu.SemaphoreType.DMA,
            pltpu.SemaphoreType.DMA,
            pltpu.SemaphoreType.DMA,
            pltpu.SemaphoreType.DMA,
            pltpu.SemaphoreType.DMA,
        ],
        compiler_params=_sc_compiler_params(),
    )
    def k(ids_hbm, whs_hbm, eler_hbm, ee_hbm, z64_hbm, z16_hbm,
          outp_hbm, den_hbm,
          eler_v, ee_v, ids0, ids1, srcw0, srcw1, dsti0, dsti1,
          rows0, rows1, w0, w1, out_sh, den_sh,
          si0, si1, sg0, sg1, so0, so1, sd0, sd1):
        cid = lax.axis_index("c")
        sid = lax.axis_index("s")
        r0 = sid * ROWS_PER_SUB
        is0 = cid == 0

        pltpu.sync_copy(z64_hbm.at[pl.ds(r0, ROWS_PER_SUB)],
                        out_sh.at[pl.ds(r0, ROWS_PER_SUB)])
        pltpu.sync_copy(z16_hbm.at[pl.ds(r0, ROWS_PER_SUB)],
                        den_sh.at[pl.ds(r0, ROWS_PER_SUB)])
        pltpu.sync_copy(eler_hbm, eler_v)
        pltpu.sync_copy(ee_hbm, ee_v)
        zero16 = jnp.zeros((LANES,), jnp.float32)

        @pl.loop(0, EK)
        def _(kk):
            w0[kk, :] = zero16
            w1[kk, :] = zero16

        plsc.subcore_barrier()

        iota16 = lax.iota(jnp.int32, 16)
        hbase = cid * 2
        whoff = cid * NPAD
        tile_base = sid * N_CHUNKS * IDS_BLK

        bufs = ((ids0, srcw0, dsti0, rows0, w0, si0, sg0, so0, sd0),
                (ids1, srcw1, dsti1, rows1, w1, si1, sg1, so1, sd1))

        pltpu.async_copy(ids_hbm.at[pl.ds(tile_base, IDS_BLK)], ids0, si0)

        @pl.loop(0, N_CHUNKS // 2)
        def _(j):
            for p in (0, 1):
                ids_v, srcw_v, dsti_v, rows_v, w_v, si, sg, so, sd = bufs[p]
                n_ids, n_si = bufs[1 - p][0], bufs[1 - p][5]
                g = j * 2 + p
                pltpu.make_async_copy(
                    ids_hbm.at[pl.ds(0, IDS_BLK)], ids_v, si).wait()
                pltpu.async_copy(
                    ids_hbm.at[pl.ds(tile_base + (g + 1) * IDS_BLK, IDS_BLK)],
                    n_ids, n_si)
                @pl.when(j > 0)
                def _():
                    pltpu.make_async_copy(
                        rows_v, out_sh.at[dsti_v], so).wait()
                    pltpu.make_async_copy(
                        w_v, den_sh.at[dsti_v], sd).wait()

                @pl.loop(0, EK // LANES)
                def _(q):
                    sl = pl.ds(q * LANES, LANES)
                    srcw_v[sl] = ids_v[pl.ds(q * LANES, LANES)] + whoff
                    dsti_v[sl] = ids_v[pl.ds(EK + q * LANES, LANES)]

                gat = pltpu.async_copy(whs_hbm.at[srcw_v], rows_v, sg)

                @pl.loop(0, EK // LANES)
                def _(q):
                    s16 = ids_v[pl.ds(q * LANES, LANES)] * 4 + hbase
                    d16 = ids_v[pl.ds(EK + q * LANES, LANES)] * 4 + hbase
                    r16 = ids_v[pl.ds(2 * EK + q * LANES, LANES)] * 4 + hbase
                    for hh in range(2):
                        sw = plsc.load_gather(eler_v, [s16 + hh])
                        dw = plsc.load_gather(eler_v, [d16 + hh])
                        ee = plsc.load_gather(ee_v, [r16 + hh])
                        el = plsc.bitcast(sw & jnp.int32(-65536), jnp.float32)
                        er = plsc.bitcast(dw << 16, jnp.float32)
                        e = el + er + ee
                        e = jnp.maximum(e, e * NEG_SLOPE)
                        w = jnp.exp(e)
                        plsc.store_scatter(
                            w_v,
                            [q * LANES + iota16,
                             jnp.full((LANES,), hh, jnp.int32)],
                            w)

                pltpu.async_copy(w_v, den_sh.at[dsti_v], sd, add=True)

                gat.wait()

                @plsc.parallel_loop(0, EK, unroll=4)
                def _(kk):
                    w16 = w_v[kk, :]
                    for cc in range(HHALF // LANES):
                        m = jnp.full((LANES,), w16[cc // 2])
                        sl = pl.ds(cc * LANES, LANES)
                        rows_v[kk, sl] = rows_v[kk, sl] * m

                pltpu.async_copy(rows_v, out_sh.at[dsti_v], so, add=True)

        pltpu.make_async_copy(ids_hbm.at[pl.ds(0, IDS_BLK)], ids0, si0).wait()
        for p in (0, 1):
            ids_v, srcw_v, dsti_v, rows_v, w_v, si, sg, so, sd = bufs[p]
            pltpu.make_async_copy(rows_v, out_sh.at[dsti_v], so).wait()
            pltpu.make_async_copy(w_v, den_sh.at[dsti_v], sd).wait()

        plsc.subcore_barrier()
        pltpu.sync_copy(out_sh.at[pl.ds(r0, ROWS_PER_SUB)],
                        outp_hbm.at[cid, pl.ds(r0, ROWS_PER_SUB)])
        pltpu.sync_copy(den_sh.at[pl.ds(r0, ROWS_PER_SUB)],
                        den_hbm.at[cid, pl.ds(r0, ROWS_PER_SUB)])

    return k(ids_packed, whs_flat, eler_flat, ee_flat, z64, z16)


def _dot(a, b):
    return jnp.dot(a, b, preferred_element_type=jnp.float32)


def _pack_eler(eler):
    eb = lax.bitcast_convert_type(eler, jnp.int32)
    el_b = eb[:, :HEADS] & jnp.int32(-65536)
    er_b = lax.shift_right_logical(eb[:, HEADS:], 16)
    return el_b | er_b


def _tc_prep(h, W, wcat, rel_table, wrcat):
    def body(h_ref, w_ref, wcat_ref, rt_ref, wrcat_ref,
             whs_ref, eler_ref, ee_ref):
        hh = h_ref[...]
        whs_ref[0] = _dot(hh, w_ref[:, :HHALF])
        whs_ref[1] = _dot(hh, w_ref[:, HHALF:])
        eler_ref[...] = _pack_eler(_dot(hh, wcat_ref[...]))
        ee_ref[...] = _dot(rt_ref[...], wrcat_ref[...])

    return pl.pallas_call(
        body,
        out_shape=(
            jax.ShapeDtypeStruct((NC, NPAD, HHALF), jnp.float32),
            jax.ShapeDtypeStruct((NPAD, HEADS), jnp.int32),
            jax.ShapeDtypeStruct((N_REL, HEADS), jnp.float32),
        ),
    )(h, W, wcat, rel_table, wrcat)


def _combine_norm(o_ref, d_ref, exp_ref):
    s = jnp.concatenate([o_ref[0], o_ref[1]], axis=1)
    den4 = jnp.concatenate([d_ref[0][:, :2], d_ref[1][:, :2]], axis=1)
    inv = 1.0 / (den4 + 1e-9)
    return s * _dot(inv, exp_ref[...])


def _tc_norm_prep(outp, den, W, wcat, rel_table, wrcat, expand):
    def body(o_ref, d_ref, w_ref, wcat_ref, rt_ref, wrcat_ref, exp_ref,
             whs_ref, eler_ref, ee_ref):
        h2 = _combine_norm(o_ref, d_ref, exp_ref)
        whs_ref[0] = _dot(h2, w_ref[:, :HHALF])
        whs_ref[1] = _dot(h2, w_ref[:, HHALF:])
        eler_ref[...] = _pack_eler(_dot(h2, wcat_ref[...]))
        ee_ref[...] = _dot(rt_ref[...], wrcat_ref[...])

    return pl.pallas_call(
        body,
        out_shape=(
            jax.ShapeDtypeStruct((NC, NPAD, HHALF), jnp.float32),
            jax.ShapeDtypeStruct((NPAD, HEADS), jnp.int32),
            jax.ShapeDtypeStruct((N_REL, HEADS), jnp.float32),
        ),
    )(outp, den, W, wcat, rel_table, wrcat, expand)


def _tc_final(outp, den, cls_idx, expand):
    n_cls = cls_idx.shape[0]

    def body(o_ref, d_ref, cls_ref, exp_ref, out_ref):
        h3 = _combine_norm(o_ref, d_ref, exp_ref)
        ids = cls_ref[...]
        col = lax.broadcasted_iota(jnp.int32, (n_cls, NPAD), 1)
        onehot = (ids[:, None] == col).astype(jnp.float32)
        out_ref[...] = _dot(onehot, h3)

    return pl.pallas_call(
        body,
        out_shape=jax.ShapeDtypeStruct((n_cls, HID), jnp.float32),
    )(outp, den, cls_idx, expand)


def _fold_attn(W, a_src, a_dst):
    Wr3 = W.reshape(W.shape[0], HEADS, D_HEAD)
    vsrc = jnp.einsum("khd,hd->kh", Wr3, a_src)
    vdst = jnp.einsum("khd,hd->kh", Wr3, a_dst)
    return jnp.concatenate([vsrc, vdst], axis=1)


def _fold_rel(Wr, a_rel):
    return jnp.einsum("khd,hd->kh", Wr.reshape(Wr.shape[0], HEADS, D_HEAD), a_rel)


_EXPAND = np.zeros((HEADS, HID), np.float32)
for _h in range(HEADS):
    _EXPAND[_h, _h * D_HEAD:(_h + 1) * D_HEAD] = 1.0


def kernel(ent_table, rel_table, W1, Wr1, a_src1, a_dst1, a_rel1,
           W2, Wr2, a_src2, a_dst2, a_rel2,
           ent_ids, rel_ids, edge_index, cls_idx):
    expand = jnp.asarray(_EXPAND)
    wcat1 = _fold_attn(W1, a_src1, a_dst1)
    wcat2 = _fold_attn(W2, a_src2, a_dst2)
    wrcat1 = _fold_rel(Wr1, a_rel1)
    wrcat2 = _fold_rel(Wr2, a_rel2)

    ids = jnp.stack([edge_index[0], edge_index[1], rel_ids])
    ids = ids.reshape(3, NS, N_CHUNKS, EK).transpose(1, 2, 0, 3).reshape(-1)
    ids = jnp.concatenate([ids, jnp.zeros((IDS_BLK,), jnp.int32)])
    z64 = jnp.zeros((NPAD, HHALF), jnp.float32)
    z16 = jnp.zeros((NPAD, LANES), jnp.float32)

    ids_pad = jnp.pad(ent_ids, (0, NPAD - N_NODES))
    h = _sc_gather_rows(ent_table, ids_pad)

    whs1, eler1, ee1 = _tc_prep(h, W1, wcat1, rel_table, wrcat1)
    outp1, den1 = _sc_edge_pass(ids, whs1.reshape(NC * NPAD, HHALF),
                                eler1.reshape(-1), ee1.reshape(-1), z64, z16)

    whs2, eler2, ee2 = _tc_norm_prep(outp1, den1, W2, wcat2, rel_table, wrcat2,
                                     expand)
    outp2, den2 = _sc_edge_pass(ids, whs2.reshape(NC * NPAD, HHALF),
                                eler2.reshape(-1), ee2.reshape(-1), z64, z16)

    return _tc_final(outp2, den2, cls_idx, expand)


# speedup vs baseline: 81.0135x; 1.0005x over previous
import dataclasses
import functools

import numpy as np
import jax
import jax.numpy as jnp
from jax import lax
from jax.experimental import pallas as pl
from jax.experimental.pallas import tpu as pltpu
from jax.experimental.pallas import tpu_sc as plsc

N_NODES = 10000
N_EDGES = 320000
N_REL = 1000
HID = 128
HEADS = 4
D_HEAD = HID // HEADS
NEG_SLOPE = 0.2

NC = 2
NS = 16
LANES = 16
NW = NC * NS

HHALF = HID // NC
EK = 80
E_PER_S = N_EDGES // NS
N_CHUNKS = E_PER_S // EK

NPAD = 10240
GPW = NPAD // NW
ROWS_PER_SUB = NPAD // NS


def _vector_mesh():
    return plsc.VectorSubcoreMesh(core_axis_name="c", subcore_axis_name="s")


def _sc_compiler_params():
    cp = pltpu.CompilerParams()
    fields = pltpu.CompilerParams.__dataclass_fields__
    if "needs_layout_passes" in fields:
        cp = dataclasses.replace(cp, needs_layout_passes=False)
    if "use_tc_tiling_on_sc" in fields:
        cp = dataclasses.replace(cp, use_tc_tiling_on_sc=False)
    return cp


@jax.jit
def _sc_gather_rows(table, idx):
    n_sub = GPW // EK

    @functools.partial(
        pl.kernel,
        out_type=jax.ShapeDtypeStruct((NPAD, HID), jnp.float32),
        mesh=_vector_mesh(),
        scratch_types=[
            pltpu.VMEM((EK,), jnp.int32),
            pltpu.VMEM((EK, HID), jnp.float32),
            pltpu.SemaphoreType.DMA,
        ],
    )
    def k(table_hbm, idx_hbm, out_hbm, idx_v, rows_v, sem):
        wid = lax.axis_index("s") * NC + lax.axis_index("c")

        @pl.loop(0, n_sub)
        def _(g):
            base = wid * GPW + g * EK
            pltpu.sync_copy(idx_hbm.at[pl.ds(base, EK)], idx_v)
            pltpu.async_copy(table_hbm.at[idx_v], rows_v, sem).wait()
            pltpu.sync_copy(rows_v, out_hbm.at[pl.ds(base, EK)])

    return k(table, idx)


IDS_BLK = 3 * EK


@jax.jit
def _sc_edge_pass(ids_packed, whs_flat, eler_flat, ee_flat, z64, z16):
    @functools.partial(
        pl.kernel,
        out_type=(
            jax.ShapeDtypeStruct((NC, NPAD, HHALF), jnp.float32),
            jax.ShapeDtypeStruct((NC, NPAD, LANES), jnp.float32),
        ),
        mesh=_vector_mesh(),
        scratch_types=[
            pltpu.VMEM((NPAD * 4,), jnp.int32),
            pltpu.VMEM((N_REL * 4,), jnp.float32),
            pltpu.VMEM((IDS_BLK,), jnp.int32),
            pltpu.VMEM((IDS_BLK,), jnp.int32),
            pltpu.VMEM((EK,), jnp.int32),
            pltpu.VMEM((EK,), jnp.int32),
            pltpu.VMEM((EK,), jnp.int32),
            pltpu.VMEM((EK,), jnp.int32),
            pltpu.VMEM((EK, HHALF), jnp.float32),
            pltpu.VMEM((EK, HHALF), jnp.float32),
            pltpu.VMEM((EK, LANES), jnp.float32),
            pltpu.VMEM((EK, LANES), jnp.float32),
            pltpu.VMEM_SHARED((NPAD, HHALF), jnp.float32),
            pltpu.VMEM_SHARED((NPAD, LANES), jnp.float32),
            pltpu.SemaphoreType.DMA,
            pltpu.SemaphoreType.DMA,
            pltpu.SemaphoreType.DMA,
            pltpu.SemaphoreType.DMA,
            pltpu.SemaphoreType.DMA,
            pltpu.SemaphoreType.DMA,
            pltpu.SemaphoreType.DMA,
            pltpu.SemaphoreType.DMA,
        ],
        compiler_params=_sc_compiler_params(),
    )
    def k(ids_hbm, whs_hbm, eler_hbm, ee_hbm, z64_hbm, z16_hbm,
          outp_hbm, den_hbm,
          eler_v, ee_v, ids0, ids1, srcw0, srcw1, dsti0, dsti1,
          rows0, rows1, w0, w1, out_sh, den_sh,
          si0, si1, sg0, sg1, so0, so1, sd0, sd1):
        cid = lax.axis_index("c")
        sid = lax.axis_index("s")
        r0 = sid * ROWS_PER_SUB
        is0 = cid == 0

        pltpu.sync_copy(z64_hbm.at[pl.ds(r0, ROWS_PER_SUB)],
                        out_sh.at[pl.ds(r0, ROWS_PER_SUB)])
        pltpu.sync_copy(z16_hbm.at[pl.ds(r0, ROWS_PER_SUB)],
                        den_sh.at[pl.ds(r0, ROWS_PER_SUB)])
        pltpu.sync_copy(eler_hbm, eler_v)
        pltpu.sync_copy(ee_hbm, ee_v)
        zero16 = jnp.zeros((LANES,), jnp.float32)

        @pl.loop(0, EK)
        def _(kk):
            w0[kk, :] = zero16
            w1[kk, :] = zero16

        plsc.subcore_barrier()

        iota16 = lax.iota(jnp.int32, 16)
        hbase = cid * 2
        whoff = cid * NPAD
        tile_base = sid * N_CHUNKS * IDS_BLK

        bufs = ((ids0, srcw0, dsti0, rows0, w0, si0, sg0, so0, sd0),
                (ids1, srcw1, dsti1, rows1, w1, si1, sg1, so1, sd1))

        pltpu.async_copy(ids_hbm.at[pl.ds(tile_base, IDS_BLK)], ids0, si0)

        @pl.loop(0, N_CHUNKS // 2)
        def _(j):
            for p in (0, 1):
                ids_v, srcw_v, dsti_v, rows_v, w_v, si, sg, so, sd = bufs[p]
                n_ids, n_si = bufs[1 - p][0], bufs[1 - p][5]
                g = j * 2 + p
                pltpu.make_async_copy(
                    ids_hbm.at[pl.ds(0, IDS_BLK)], ids_v, si).wait()
                pltpu.async_copy(
                    ids_hbm.at[pl.ds(tile_base + (g + 1) * IDS_BLK, IDS_BLK)],
                    n_ids, n_si)
                @pl.when(j > 0)
                def _():
                    pltpu.make_async_copy(
                        rows_v, out_sh.at[dsti_v], so).wait()
                    pltpu.make_async_copy(
                        w_v, den_sh.at[dsti_v], sd).wait()

                @plsc.parallel_loop(0, EK // LANES, unroll=5)
                def _(q):
                    sl = pl.ds(q * LANES, LANES)
                    srcw_v[sl] = ids_v[pl.ds(q * LANES, LANES)] + whoff
                    dsti_v[sl] = ids_v[pl.ds(EK + q * LANES, LANES)]

                gat = pltpu.async_copy(whs_hbm.at[srcw_v], rows_v, sg)

                @plsc.parallel_loop(0, EK // LANES, unroll=5)
                def _(q):
                    s16 = ids_v[pl.ds(q * LANES, LANES)] * 4 + hbase
                    d16 = ids_v[pl.ds(EK + q * LANES, LANES)] * 4 + hbase
                    r16 = ids_v[pl.ds(2 * EK + q * LANES, LANES)] * 4 + hbase
                    for hh in range(2):
                        sw = plsc.load_gather(eler_v, [s16 + hh])
                        dw = plsc.load_gather(eler_v, [d16 + hh])
                        ee = plsc.load_gather(ee_v, [r16 + hh])
                        el = plsc.bitcast(sw & jnp.int32(-65536), jnp.float32)
                        er = plsc.bitcast(dw << 16, jnp.float32)
                        e = el + er + ee
                        e = jnp.maximum(e, e * NEG_SLOPE)
                        w = jnp.exp(e)
                        plsc.store_scatter(
                            w_v,
                            [q * LANES + iota16,
                             jnp.full((LANES,), hh, jnp.int32)],
                            w)

                pltpu.async_copy(w_v, den_sh.at[dsti_v], sd, add=True)

                gat.wait()

                @plsc.parallel_loop(0, EK, unroll=8)
                def _(kk):
                    w16 = w_v[kk, :]
                    for cc in range(HHALF // LANES):
                        m = jnp.full((LANES,), w16[cc // 2])
                        sl = pl.ds(cc * LANES, LANES)
                        rows_v[kk, sl] = rows_v[kk, sl] * m

                pltpu.async_copy(rows_v, out_sh.at[dsti_v], so, add=True)

        pltpu.make_async_copy(ids_hbm.at[pl.ds(0, IDS_BLK)], ids0, si0).wait()
        for p in (0, 1):
            ids_v, srcw_v, dsti_v, rows_v, w_v, si, sg, so, sd = bufs[p]
            pltpu.make_async_copy(rows_v, out_sh.at[dsti_v], so).wait()
            pltpu.make_async_copy(w_v, den_sh.at[dsti_v], sd).wait()

        plsc.subcore_barrier()
        pltpu.sync_copy(out_sh.at[pl.ds(r0, ROWS_PER_SUB)],
                        outp_hbm.at[cid, pl.ds(r0, ROWS_PER_SUB)])
        pltpu.sync_copy(den_sh.at[pl.ds(r0, ROWS_PER_SUB)],
                        den_hbm.at[cid, pl.ds(r0, ROWS_PER_SUB)])

    return k(ids_packed, whs_flat, eler_flat, ee_flat, z64, z16)


def _dot(a, b):
    return jnp.dot(a, b, preferred_element_type=jnp.float32)


def _pack_eler(eler):
    eb = lax.bitcast_convert_type(eler, jnp.int32)
    el_b = eb[:, :HEADS] & jnp.int32(-65536)
    er_b = lax.shift_right_logical(eb[:, HEADS:], 16)
    return el_b | er_b


def _tc_prep(h, W, wcat, rel_table, wrcat):
    def body(h_ref, w_ref, wcat_ref, rt_ref, wrcat_ref,
             whs_ref, eler_ref, ee_ref):
        hh = h_ref[...]
        whs_ref[0] = _dot(hh, w_ref[:, :HHALF])
        whs_ref[1] = _dot(hh, w_ref[:, HHALF:])
        eler_ref[...] = _pack_eler(_dot(hh, wcat_ref[...]))
        ee_ref[...] = _dot(rt_ref[...], wrcat_ref[...])

    return pl.pallas_call(
        body,
        out_shape=(
            jax.ShapeDtypeStruct((NC, NPAD, HHALF), jnp.float32),
            jax.ShapeDtypeStruct((NPAD, HEADS), jnp.int32),
            jax.ShapeDtypeStruct((N_REL, HEADS), jnp.float32),
        ),
    )(h, W, wcat, rel_table, wrcat)


def _combine_norm(o_ref, d_ref, exp_ref):
    s = jnp.concatenate([o_ref[0], o_ref[1]], axis=1)
    den4 = jnp.concatenate([d_ref[0][:, :2], d_ref[1][:, :2]], axis=1)
    inv = 1.0 / (den4 + 1e-9)
    return s * _dot(inv, exp_ref[...])


def _tc_norm_prep(outp, den, W, wcat, rel_table, wrcat, expand):
    def body(o_ref, d_ref, w_ref, wcat_ref, rt_ref, wrcat_ref, exp_ref,
             whs_ref, eler_ref, ee_ref):
        h2 = _combine_norm(o_ref, d_ref, exp_ref)
        whs_ref[0] = _dot(h2, w_ref[:, :HHALF])
        whs_ref[1] = _dot(h2, w_ref[:, HHALF:])
        eler_ref[...] = _pack_eler(_dot(h2, wcat_ref[...]))
        ee_ref[...] = _dot(rt_ref[...], wrcat_ref[...])

    return pl.pallas_call(
        body,
        out_shape=(
            jax.ShapeDtypeStruct((NC, NPAD, HHALF), jnp.float32),
            jax.ShapeDtypeStruct((NPAD, HEADS), jnp.int32),
            jax.ShapeDtypeStruct((N_REL, HEADS), jnp.float32),
        ),
    )(outp, den, W, wcat, rel_table, wrcat, expand)


def _tc_final(outp, den, cls_idx, expand):
    n_cls = cls_idx.shape[0]

    def body(o_ref, d_ref, cls_ref, exp_ref, out_ref):
        h3 = _combine_norm(o_ref, d_ref, exp_ref)
        ids = cls_ref[...]
        col = lax.broadcasted_iota(jnp.int32, (n_cls, NPAD), 1)
        onehot = (ids[:, None] == col).astype(jnp.float32)
        out_ref[...] = _dot(onehot, h3)

    return pl.pallas_call(
        body,
        out_shape=jax.ShapeDtypeStruct((n_cls, HID), jnp.float32),
    )(outp, den, cls_idx, expand)


def _fold_attn(W, a_src, a_dst):
    Wr3 = W.reshape(W.shape[0], HEADS, D_HEAD)
    vsrc = jnp.einsum("khd,hd->kh", Wr3, a_src)
    vdst = jnp.einsum("khd,hd->kh", Wr3, a_dst)
    return jnp.concatenate([vsrc, vdst], axis=1)


def _fold_rel(Wr, a_rel):
    return jnp.einsum("khd,hd->kh", Wr.reshape(Wr.shape[0], HEADS, D_HEAD), a_rel)


_EXPAND = np.zeros((HEADS, HID), np.float32)
for _h in range(HEADS):
    _EXPAND[_h, _h * D_HEAD:(_h + 1) * D_HEAD] = 1.0


def kernel(ent_table, rel_table, W1, Wr1, a_src1, a_dst1, a_rel1,
           W2, Wr2, a_src2, a_dst2, a_rel2,
           ent_ids, rel_ids, edge_index, cls_idx):
    expand = jnp.asarray(_EXPAND)
    wcat1 = _fold_attn(W1, a_src1, a_dst1)
    wcat2 = _fold_attn(W2, a_src2, a_dst2)
    wrcat1 = _fold_rel(Wr1, a_rel1)
    wrcat2 = _fold_rel(Wr2, a_rel2)

    ids = jnp.stack([edge_index[0], edge_index[1], rel_ids])
    ids = ids.reshape(3, NS, N_CHUNKS, EK).transpose(1, 2, 0, 3).reshape(-1)
    ids = jnp.concatenate([ids, jnp.zeros((IDS_BLK,), jnp.int32)])
    z64 = jnp.zeros((NPAD, HHALF), jnp.float32)
    z16 = jnp.zeros((NPAD, LANES), jnp.float32)

    ids_pad = jnp.pad(ent_ids, (0, NPAD - N_NODES))
    h = _sc_gather_rows(ent_table, ids_pad)

    whs1, eler1, ee1 = _tc_prep(h, W1, wcat1, rel_table, wrcat1)
    outp1, den1 = _sc_edge_pass(ids, whs1.reshape(NC * NPAD, HHALF),
                                eler1.reshape(-1), ee1.reshape(-1), z64, z16)

    whs2, eler2, ee2 = _tc_norm_prep(outp1, den1, W2, wcat2, rel_table, wrcat2,
                                     expand)
    outp2, den2 = _sc_edge_pass(ids, whs2.reshape(NC * NPAD, HHALF),
                                eler2.reshape(-1), ee2.reshape(-1), z64, z16)

    return _tc_final(outp2, den2, cls_idx, expand)


# speedup vs baseline: 113.7100x; 1.4036x over previous
import dataclasses
import functools

import numpy as np
import jax
import jax.numpy as jnp
from jax import lax
from jax.experimental import pallas as pl
from jax.experimental.pallas import tpu as pltpu
from jax.experimental.pallas import tpu_sc as plsc

N_NODES = 10000
N_EDGES = 320000
N_REL = 1000
HID = 128
HEADS = 4
D_HEAD = HID // HEADS
NEG_SLOPE = 0.2

NC = 2
NS = 16
LANES = 16
NW = NC * NS

HHALF = HID // NC
EK = 80
E_PER_S = N_EDGES // NS
N_CHUNKS = E_PER_S // EK

NPAD = 10240
GPW = NPAD // NW
ROWS_PER_SUB = NPAD // NS


def _vector_mesh():
    return plsc.VectorSubcoreMesh(core_axis_name="c", subcore_axis_name="s")


def _sc_compiler_params():
    cp = pltpu.CompilerParams()
    fields = pltpu.CompilerParams.__dataclass_fields__
    if "needs_layout_passes" in fields:
        cp = dataclasses.replace(cp, needs_layout_passes=False)
    if "use_tc_tiling_on_sc" in fields:
        cp = dataclasses.replace(cp, use_tc_tiling_on_sc=False)
    return cp


@jax.jit
def _sc_gather_rows(table, idx):
    n_sub = GPW // EK

    @functools.partial(
        pl.kernel,
        out_type=jax.ShapeDtypeStruct((NPAD, HID), jnp.float32),
        mesh=_vector_mesh(),
        scratch_types=[
            pltpu.VMEM((EK,), jnp.int32),
            pltpu.VMEM((EK, HID), jnp.float32),
            pltpu.SemaphoreType.DMA,
        ],
    )
    def k(table_hbm, idx_hbm, out_hbm, idx_v, rows_v, sem):
        wid = lax.axis_index("s") * NC + lax.axis_index("c")

        @pl.loop(0, n_sub)
        def _(g):
            base = wid * GPW + g * EK
            pltpu.sync_copy(idx_hbm.at[pl.ds(base, EK)], idx_v)
            pltpu.async_copy(table_hbm.at[idx_v], rows_v, sem).wait()
            pltpu.sync_copy(rows_v, out_hbm.at[pl.ds(base, EK)])

    return k(table, idx)


IDS_BLK = 3 * EK


@jax.jit
def _sc_edge_pass(ids_packed, whs_flat, eler_flat, ee_flat, z64, z16):
    @functools.partial(
        pl.kernel,
        out_type=(
            jax.ShapeDtypeStruct((NC, NPAD, HHALF), jnp.float32),
            jax.ShapeDtypeStruct((NC, NPAD, LANES), jnp.float32),
        ),
        mesh=_vector_mesh(),
        scratch_types=[
            pltpu.VMEM((NPAD * 4,), jnp.int32),
            pltpu.VMEM((N_REL * 4,), jnp.float32),
            pltpu.VMEM((IDS_BLK,), jnp.int32),
            pltpu.VMEM((IDS_BLK,), jnp.int32),
            pltpu.VMEM((EK,), jnp.int32),
            pltpu.VMEM((EK,), jnp.int32),
            pltpu.VMEM((EK,), jnp.int32),
            pltpu.VMEM((EK,), jnp.int32),
            pltpu.VMEM((EK, HHALF), jnp.float32),
            pltpu.VMEM((EK, HHALF), jnp.float32),
            pltpu.VMEM((EK, LANES), jnp.float32),
            pltpu.VMEM((EK, LANES), jnp.float32),
            pltpu.VMEM_SHARED((NPAD, HHALF), jnp.float32),
            pltpu.VMEM_SHARED((NPAD, LANES), jnp.float32),
            pltpu.SemaphoreType.DMA,
            pltpu.SemaphoreType.DMA,
            pltpu.SemaphoreType.DMA,
            pltpu.SemaphoreType.DMA,
            pltpu.SemaphoreType.DMA,
            pltpu.SemaphoreType.DMA,
            pltpu.SemaphoreType.DMA,
            pltpu.SemaphoreType.DMA,
        ],
        compiler_params=_sc_compiler_params(),
    )
    def k(ids_hbm, whs_hbm, eler_hbm, ee_hbm, z64_hbm, z16_hbm,
          outp_hbm, den_hbm,
          eler_v, ee_v, ids0, ids1, srcw0, srcw1, dsti0, dsti1,
          rows0, rows1, w0, w1, out_sh, den_sh,
          si0, si1, sg0, sg1, so0, so1, sd0, sd1):
        cid = lax.axis_index("c")
        sid = lax.axis_index("s")
        r0 = sid * ROWS_PER_SUB
        is0 = cid == 0

        pltpu.sync_copy(z64_hbm.at[pl.ds(r0, ROWS_PER_SUB)],
                        out_sh.at[pl.ds(r0, ROWS_PER_SUB)])
        pltpu.sync_copy(z16_hbm.at[pl.ds(r0, ROWS_PER_SUB)],
                        den_sh.at[pl.ds(r0, ROWS_PER_SUB)])
        pltpu.sync_copy(eler_hbm, eler_v)
        pltpu.sync_copy(ee_hbm, ee_v)
        zero16 = jnp.zeros((LANES,), jnp.float32)

        @pl.loop(0, EK)
        def _(kk):
            w0[kk, :] = zero16
            w1[kk, :] = zero16

        plsc.subcore_barrier()

        iota16 = lax.iota(jnp.int32, 16)
        hbase = cid * 2
        whoff = cid * NPAD
        tile_base = sid * N_CHUNKS * IDS_BLK

        bufs = ((ids0, srcw0, dsti0, rows0, w0, si0, sg0, so0, sd0),
                (ids1, srcw1, dsti1, rows1, w1, si1, sg1, so1, sd1))

        pltpu.async_copy(ids_hbm.at[pl.ds(tile_base, IDS_BLK)], ids0, si0)

        @pl.loop(0, N_CHUNKS // 2)
        def _(j):
            for p in (0, 1):
                ids_v, srcw_v, dsti_v, rows_v, w_v, si, sg, so, sd = bufs[p]
                n_ids, n_si = bufs[1 - p][0], bufs[1 - p][5]
                g = j * 2 + p
                pltpu.make_async_copy(
                    ids_hbm.at[pl.ds(0, IDS_BLK)], ids_v, si).wait()
                pltpu.async_copy(
                    ids_hbm.at[pl.ds(tile_base + (g + 1) * IDS_BLK, IDS_BLK)],
                    n_ids, n_si)
                @pl.when(j > 0)
                def _():
                    pltpu.make_async_copy(
                        rows_v, out_sh.at[dsti_v], so).wait()
                    pltpu.make_async_copy(
                        w_v, den_sh.at[dsti_v], sd).wait()

                @plsc.parallel_loop(0, EK // LANES, unroll=5)
                def _(q):
                    sl = pl.ds(q * LANES, LANES)
                    srcw_v[sl] = ids_v[pl.ds(q * LANES, LANES)] + whoff
                    dsti_v[sl] = ids_v[pl.ds(EK + q * LANES, LANES)]

                pltpu.async_copy(whs_hbm.at[srcw_v], rows_v, sg)

                @plsc.parallel_loop(0, EK // LANES, unroll=5)
                def _(q):
                    s16 = ids_v[pl.ds(q * LANES, LANES)] * 4 + hbase
                    d16 = ids_v[pl.ds(EK + q * LANES, LANES)] * 4 + hbase
                    r16 = ids_v[pl.ds(2 * EK + q * LANES, LANES)] * 4 + hbase
                    for hh in range(2):
                        sw = plsc.load_gather(eler_v, [s16 + hh])
                        dw = plsc.load_gather(eler_v, [d16 + hh])
                        ee = plsc.load_gather(ee_v, [r16 + hh])
                        el = plsc.bitcast(sw & jnp.int32(-65536), jnp.float32)
                        er = plsc.bitcast(dw << 16, jnp.float32)
                        e = el + er + ee
                        e = jnp.maximum(e, e * NEG_SLOPE)
                        w = jnp.exp(e)
                        plsc.store_scatter(
                            w_v,
                            [q * LANES + iota16,
                             jnp.full((LANES,), hh, jnp.int32)],
                            w)

                pltpu.async_copy(w_v, den_sh.at[dsti_v], sd, add=True)

                p_ids, p_srcw, p_dsti, p_rows, p_w, p_si, p_sg, p_so, p_sd = \
                    bufs[1 - p]

                def scale_and_scatter():
                    pltpu.make_async_copy(
                        whs_hbm.at[p_srcw], p_rows, p_sg).wait()

                    @plsc.parallel_loop(0, EK, unroll=8)
                    def _(kk):
                        w16 = p_w[kk, :]
                        for cc in range(HHALF // LANES):
                            m = jnp.full((LANES,), w16[cc // 2])
                            sl = pl.ds(cc * LANES, LANES)
                            p_rows[kk, sl] = p_rows[kk, sl] * m

                    pltpu.async_copy(p_rows, out_sh.at[p_dsti], p_so, add=True)

                if p == 0:
                    @pl.when(j > 0)
                    def _():
                        scale_and_scatter()
                else:
                    scale_and_scatter()

        pltpu.make_async_copy(
            whs_hbm.at[bufs[1][1]], bufs[1][3], bufs[1][6]).wait()

        @plsc.parallel_loop(0, EK, unroll=8)
        def _(kk):
            w16 = bufs[1][4][kk, :]
            for cc in range(HHALF // LANES):
                m = jnp.full((LANES,), w16[cc // 2])
                sl = pl.ds(cc * LANES, LANES)
                bufs[1][3][kk, sl] = bufs[1][3][kk, sl] * m

        pltpu.async_copy(bufs[1][3], out_sh.at[bufs[1][2]], bufs[1][7],
                         add=True)
        pltpu.make_async_copy(ids_hbm.at[pl.ds(0, IDS_BLK)], ids0, si0).wait()
        for p in (0, 1):
            ids_v, srcw_v, dsti_v, rows_v, w_v, si, sg, so, sd = bufs[p]
            pltpu.make_async_copy(rows_v, out_sh.at[dsti_v], so).wait()
            pltpu.make_async_copy(w_v, den_sh.at[dsti_v], sd).wait()

        plsc.subcore_barrier()
        pltpu.sync_copy(out_sh.at[pl.ds(r0, ROWS_PER_SUB)],
                        outp_hbm.at[cid, pl.ds(r0, ROWS_PER_SUB)])
        pltpu.sync_copy(den_sh.at[pl.ds(r0, ROWS_PER_SUB)],
                        den_hbm.at[cid, pl.ds(r0, ROWS_PER_SUB)])

    return k(ids_packed, whs_flat, eler_flat, ee_flat, z64, z16)


def _dot(a, b):
    return jnp.dot(a, b, preferred_element_type=jnp.float32)


def _pack_eler(eler):
    eb = lax.bitcast_convert_type(eler, jnp.int32)
    el_b = eb[:, :HEADS] & jnp.int32(-65536)
    er_b = lax.shift_right_logical(eb[:, HEADS:], 16)
    return el_b | er_b


def _tc_prep(h, W, wcat, rel_table, wrcat):
    def body(h_ref, w_ref, wcat_ref, rt_ref, wrcat_ref,
             whs_ref, eler_ref, ee_ref):
        hh = h_ref[...]
        whs_ref[0] = _dot(hh, w_ref[:, :HHALF])
        whs_ref[1] = _dot(hh, w_ref[:, HHALF:])
        eler_ref[...] = _pack_eler(_dot(hh, wcat_ref[...]))
        ee_ref[...] = _dot(rt_ref[...], wrcat_ref[...])

    return pl.pallas_call(
        body,
        out_shape=(
            jax.ShapeDtypeStruct((NC, NPAD, HHALF), jnp.float32),
            jax.ShapeDtypeStruct((NPAD, HEADS), jnp.int32),
            jax.ShapeDtypeStruct((N_REL, HEADS), jnp.float32),
        ),
    )(h, W, wcat, rel_table, wrcat)


def _combine_norm(o_ref, d_ref, exp_ref):
    s = jnp.concatenate([o_ref[0], o_ref[1]], axis=1)
    den4 = jnp.concatenate([d_ref[0][:, :2], d_ref[1][:, :2]], axis=1)
    inv = 1.0 / (den4 + 1e-9)
    return s * _dot(inv, exp_ref[...])


def _tc_norm_prep(outp, den, W, wcat, rel_table, wrcat, expand):
    def body(o_ref, d_ref, w_ref, wcat_ref, rt_ref, wrcat_ref, exp_ref,
             whs_ref, eler_ref, ee_ref):
        h2 = _combine_norm(o_ref, d_ref, exp_ref)
        whs_ref[0] = _dot(h2, w_ref[:, :HHALF])
        whs_ref[1] = _dot(h2, w_ref[:, HHALF:])
        eler_ref[...] = _pack_eler(_dot(h2, wcat_ref[...]))
        ee_ref[...] = _dot(rt_ref[...], wrcat_ref[...])

    return pl.pallas_call(
        body,
        out_shape=(
            jax.ShapeDtypeStruct((NC, NPAD, HHALF), jnp.float32),
            jax.ShapeDtypeStruct((NPAD, HEADS), jnp.int32),
            jax.ShapeDtypeStruct((N_REL, HEADS), jnp.float32),
        ),
    )(outp, den, W, wcat, rel_table, wrcat, expand)


def _tc_final(outp, den, cls_idx, expand):
    n_cls = cls_idx.shape[0]

    def body(o_ref, d_ref, cls_ref, exp_ref, out_ref):
        h3 = _combine_norm(o_ref, d_ref, exp_ref)
        ids = cls_ref[...]
        col = lax.broadcasted_iota(jnp.int32, (n_cls, NPAD), 1)
        onehot = (ids[:, None] == col).astype(jnp.float32)
        out_ref[...] = _dot(onehot, h3)

    return pl.pallas_call(
        body,
        out_shape=jax.ShapeDtypeStruct((n_cls, HID), jnp.float32),
    )(outp, den, cls_idx, expand)


def _fold_attn(W, a_src, a_dst):
    Wr3 = W.reshape(W.shape[0], HEADS, D_HEAD)
    vsrc = jnp.einsum("khd,hd->kh", Wr3, a_src)
    vdst = jnp.einsum("khd,hd->kh", Wr3, a_dst)
    return jnp.concatenate([vsrc, vdst], axis=1)


def _fold_rel(Wr, a_rel):
    return jnp.einsum("khd,hd->kh", Wr.reshape(Wr.shape[0], HEADS, D_HEAD), a_rel)


_EXPAND = np.zeros((HEADS, HID), np.float32)
for _h in range(HEADS):
    _EXPAND[_h, _h * D_HEAD:(_h + 1) * D_HEAD] = 1.0


def kernel(ent_table, rel_table, W1, Wr1, a_src1, a_dst1, a_rel1,
           W2, Wr2, a_src2, a_dst2, a_rel2,
           ent_ids, rel_ids, edge_index, cls_idx):
    expand = jnp.asarray(_EXPAND)
    wcat1 = _fold_attn(W1, a_src1, a_dst1)
    wcat2 = _fold_attn(W2, a_src2, a_dst2)
    wrcat1 = _fold_rel(Wr1, a_rel1)
    wrcat2 = _fold_rel(Wr2, a_rel2)

    ids = jnp.stack([edge_index[0], edge_index[1], rel_ids])
    ids = ids.reshape(3, NS, N_CHUNKS, EK).transpose(1, 2, 0, 3).reshape(-1)
    ids = jnp.concatenate([ids, jnp.zeros((IDS_BLK,), jnp.int32)])
    z64 = jnp.zeros((NPAD, HHALF), jnp.float32)
    z16 = jnp.zeros((NPAD, LANES), jnp.float32)

    ids_pad = jnp.pad(ent_ids, (0, NPAD - N_NODES))
    h = _sc_gather_rows(ent_table, ids_pad)

    whs1, eler1, ee1 = _tc_prep(h, W1, wcat1, rel_table, wrcat1)
    outp1, den1 = _sc_edge_pass(ids, whs1.reshape(NC * NPAD, HHALF),
                                eler1.reshape(-1), ee1.reshape(-1), z64, z16)

    whs2, eler2, ee2 = _tc_norm_prep(outp1, den1, W2, wcat2, rel_table, wrcat2,
                                     expand)
    outp2, den2 = _sc_edge_pass(ids, whs2.reshape(NC * NPAD, HHALF),
                                eler2.reshape(-1), ee2.reshape(-1), z64, z16)

    return _tc_final(outp2, den2, cls_idx, expand)
